# baseline (speedup 1.0000x reference)
import functools

import jax
import jax.numpy as jnp
from jax import lax
from jax.experimental import pallas as pl
from jax.experimental.pallas import tpu as pltpu

N_REQ = 100000
N_VEH = 100000
N_PAS = 100000
E = 1600000
B = 1024
SLOTS = 1040


def _feat_body(rx, vx, px, wr, br, wv, bv, wp, bp, ro, vo, po):
    ro[...] = jnp.tanh(
        jnp.dot(rx[...], wr[...], preferred_element_type=jnp.float32) + br[...])
    vo[...] = jnp.tanh(
        jnp.dot(vx[...], wv[...], preferred_element_type=jnp.float32) + bv[...])
    po[...] = jnp.tanh(
        jnp.dot(px[...], wp[...], preferred_element_type=jnp.float32) + bp[...])


def _features(rx, vx, px, wr, br, wv, bv, wp, bp):
    R = 2000
    n = N_REQ // R
    row_spec = lambda d: pl.BlockSpec((R, d), lambda i: (i, 0))
    full = lambda a: pl.BlockSpec(a.shape, lambda i: tuple(0 for _ in a.shape))
    args = (rx, vx, px, wr, br.reshape(1, -1), wv, bv.reshape(1, -1),
            wp, bp.reshape(1, -1))
    return pl.pallas_call(
        _feat_body,
        grid=(n,),
        in_specs=[row_spec(rx.shape[1]), row_spec(vx.shape[1]),
                  row_spec(px.shape[1])] + [full(a) for a in args[3:]],
        out_specs=[row_spec(16), row_spec(16), row_spec(16)],
        out_shape=[
            jax.ShapeDtypeStruct((N_REQ, 16), jnp.float32),
            jax.ShapeDtypeStruct((N_VEH, 16), jnp.float32),
            jax.ShapeDtypeStruct((N_PAS, 16), jnp.float32),
        ],
    )(*args)


def _final_body(rs, rc, vs, vc, ps, pc, sid, ego_v,
                w1, b1, w2, b2, w3, b3, out):
    req_feat = rs[...] / jnp.maximum(rc[...], 1.0)
    veh_mean = vs[...] / jnp.maximum(vc[...], 1.0)
    pas_mean = ps[...] / jnp.maximum(pc[...], 1.0)
    cols = lax.broadcasted_iota(jnp.int32, (B, SLOTS), 1)
    onehot = (cols == sid[...]).astype(jnp.float32)
    ego_p = jnp.dot(onehot, pas_mean, preferred_element_type=jnp.float32)
    act = jnp.concatenate([req_feat, ego_v[...], ego_p, veh_mean], axis=-1)
    h = jnp.tanh(jnp.dot(act, w1[...], preferred_element_type=jnp.float32) + b1[...])
    h = jnp.tanh(jnp.dot(h, w2[...], preferred_element_type=jnp.float32) + b2[...])
    out[...] = jnp.dot(h, w3[...], preferred_element_type=jnp.float32) + b3[...]


def _final(rs, rc, vs, vc, ps, pc, sid, ego_v, w1, b1, w2, b2, w3, b3):
    full = lambda a: pl.BlockSpec(a.shape, lambda: tuple(0 for _ in a.shape))
    args = (rs, rc, vs, vc, ps, pc, sid.reshape(B, 1), ego_v,
            w1, b1.reshape(1, -1), w2, b2.reshape(1, -1), w3, b3.reshape(1, -1))
    return pl.pallas_call(
        _final_body,
        in_specs=[full(a) for a in args],
        out_specs=pl.BlockSpec((B, 1), lambda: (0, 0)),
        out_shape=jax.ShapeDtypeStruct((B, 1), jnp.float32),
    )(*args)


def kernel(requests_x, requests_x_batch, vehicles_x, vehicles_x_batch,
           passengers_x, veh2pas_receiver_edge_index, veh2pas_sender_edge_index,
           req2veh_receiver_start_index,
           W_req, b_req, W_veh, b_veh, W_pas, b_pas,
           W_c1, b_c1, W_c2, b_c2, W_c3, b_c3):
    req_feat_full, veh_feat, pas_feat = _features(
        requests_x, vehicles_x, passengers_x,
        W_req, b_req, W_veh, b_veh, W_pas, b_pas)

    ones_req = jnp.ones((N_REQ,), jnp.float32)
    req_sum = jax.ops.segment_sum(req_feat_full, requests_x_batch, num_segments=SLOTS)
    req_cnt = jax.ops.segment_sum(ones_req, requests_x_batch, num_segments=SLOTS)
    veh_sum = jax.ops.segment_sum(veh_feat, vehicles_x_batch, num_segments=SLOTS)
    veh_cnt = jax.ops.segment_sum(ones_req, vehicles_x_batch, num_segments=SLOTS)

    r2v = req2veh_receiver_start_index
    mark = jnp.full((N_VEH,), B, jnp.int32).at[r2v].set(
        jnp.arange(B, dtype=jnp.int32))
    cid = mark[veh2pas_sender_edge_index]
    gathered = jnp.take(pas_feat, veh2pas_receiver_edge_index, axis=0)
    pas_sum = jax.ops.segment_sum(gathered, cid, num_segments=SLOTS)
    pas_cnt = jax.ops.segment_sum(jnp.ones((E,), jnp.float32), cid,
                                  num_segments=SLOTS)
    sid = mark[r2v]
    ego_v = jnp.take(veh_feat, r2v, axis=0)

    return _final(req_sum[:B], req_cnt[:B].reshape(B, 1),
                  veh_sum[:B], veh_cnt[:B].reshape(B, 1),
                  pas_sum, pas_cnt.reshape(SLOTS, 1),
                  sid, ego_v, W_c1, b_c1, W_c2, b_c2, W_c3, b_c3)
